# Initial kernel scaffold; baseline (speedup 1.0000x reference)
import functools

import jax
import jax.numpy as jnp
from jax import lax
from jax.experimental import pallas as pl
from jax.experimental.pallas import tpu as pltpu
from jax.experimental.pallas import tpu_sc as plsc

_N = 10000
_E = 320000
_HID = 200
_HP = 256
_NP = 10240
_NG = _NP // 128
_NC = 2
_NS = 16
_NW = _NC * _NS
_EPW = _E // _NW
_L = 16

_MESH = plsc.VectorSubcoreMesh(
    core_axis_name="c", subcore_axis_name="s", num_cores=_NC, num_subcores=_NS
)


def _worker_id():
    return lax.axis_index("s") * _NC + lax.axis_index("c")


def _zero_vmem(ref, n):
    zeros = jnp.zeros((_L,), jnp.float32)

    def body(i, carry):
        ref[pl.ds(i * _L, _L)] = zeros
        return carry

    lax.fori_loop(0, n // _L, body, 0)


@functools.partial(
    pl.kernel,
    out_type=jax.ShapeDtypeStruct((_NW, _NP), jnp.float32),
    mesh=_MESH,
    scratch_types=[
        pltpu.VMEM((_EPW,), jnp.int32),
        pltpu.VMEM((_NP,), jnp.float32),
    ],
)
def _sc_degree(dst_hbm, out_hbm, dst_v, acc_v):
    wid = _worker_id()
    pltpu.sync_copy(dst_hbm.at[pl.ds(wid * _EPW, _EPW)], dst_v)
    _zero_vmem(acc_v, _NP)
    ones = jnp.ones((_L,), jnp.float32)

    def body(i, carry):
        idx = dst_v[pl.ds(i * _L, _L)]
        plsc.addupdate_scatter(acc_v, [idx], ones)
        return carry

    lax.fori_loop(0, _EPW // _L, body, 0)
    pltpu.sync_copy(acc_v, out_hbm.at[wid])


@functools.partial(
    pl.kernel,
    out_type=jax.ShapeDtypeStruct((_NW, _NP), jnp.float32),
    mesh=_MESH,
    scratch_types=[
        pltpu.VMEM((_EPW,), jnp.int32),
        pltpu.VMEM((_EPW,), jnp.int32),
        pltpu.VMEM((_NP,), jnp.float32),
        pltpu.VMEM((_NP,), jnp.float32),
    ],
)
def _sc_message(src_hbm, dst_hbm, u_hbm, out_hbm, src_v, dst_v, u_v, acc_v):
    wid = _worker_id()
    pltpu.sync_copy(u_hbm, u_v)
    pltpu.sync_copy(src_hbm.at[pl.ds(wid * _EPW, _EPW)], src_v)
    pltpu.sync_copy(dst_hbm.at[pl.ds(wid * _EPW, _EPW)], dst_v)
    _zero_vmem(acc_v, _NP)

    def body(i, carry):
        sidx = src_v[pl.ds(i * _L, _L)]
        didx = dst_v[pl.ds(i * _L, _L)]
        vals = plsc.load_gather(u_v, [sidx])
        plsc.addupdate_scatter(acc_v, [didx], vals)
        return carry

    lax.fori_loop(0, _EPW // _L, body, 0)
    pltpu.sync_copy(acc_v, out_hbm.at[wid])


def _matvec_body(x_ref, w_ref, h_ref):
    h_ref[...] = lax.dot(x_ref[...], w_ref[...], preferred_element_type=jnp.float32)


def _prep_body(degp_ref, h_ref, dinv_ref, u_ref):
    deg = jnp.sum(degp_ref[...], axis=0) + 1.0
    dinv = lax.rsqrt(deg)
    dinv_ref[...] = dinv
    u_ref[...] = dinv * h_ref[...]


def _combine_body(sp_ref, dinv_ref, u_ref, bg_ref, z_ref):
    s = jnp.sum(sp_ref[...], axis=0)
    z_ref[...] = dinv_ref[...] * (s + u_ref[...]) + bg_ref[0, 0]


def _mlp_body(z_ref, w1_ref, b1_ref, w2_ref, b2_ref, w3_ref, b3_ref, w4_ref,
              b4_ref, w5_ref, b5_ref, w6_ref, b6_ref, w7_ref, b7_ref, w8_ref,
              b8_ref, o_ref):
    a = jnp.maximum(z_ref[...] * w1_ref[...] + b1_ref[...], 0.0)
    for w_ref, b_ref in ((w2_ref, b2_ref), (w3_ref, b3_ref), (w4_ref, b4_ref),
                         (w5_ref, b5_ref), (w6_ref, b6_ref), (w7_ref, b7_ref)):
        a = lax.dot(a, w_ref[...], preferred_element_type=jnp.float32)
        a = jnp.maximum(a + b_ref[...], 0.0)
    logits = lax.dot(a, w8_ref[...], preferred_element_type=jnp.float32)
    o_ref[...] = jax.nn.sigmoid(logits + b8_ref[0, 0])


def kernel(x, edge_index, W_gcn, b_gcn, mlp_Ws, mlp_bs):
    f32 = jnp.float32
    src = edge_index[0]
    dst = edge_index[1]

    x_pad = jnp.pad(x, ((0, _NP - _N), (0, 0)))

    h = pl.pallas_call(
        _matvec_body,
        out_shape=jax.ShapeDtypeStruct((_NP, 1), f32),
    )(x_pad, W_gcn)

    degp = _sc_degree(dst)

    dinv, u = pl.pallas_call(
        _prep_body,
        out_shape=(
            jax.ShapeDtypeStruct((_NG, 128), f32),
            jax.ShapeDtypeStruct((_NG, 128), f32),
        ),
    )(degp.reshape(_NW, _NG, 128), h.reshape(_NG, 128))

    sp = _sc_message(src, dst, u.reshape(_NP))

    z = pl.pallas_call(
        _combine_body,
        out_shape=jax.ShapeDtypeStruct((_NG, 128), f32),
    )(sp.reshape(_NW, _NG, 128), dinv, u, b_gcn.reshape(1, 1).astype(f32))

    w1 = jnp.pad(mlp_Ws[0], ((0, 0), (0, _HP - _HID)))
    wmid = [jnp.pad(w, ((0, _HP - _HID), (0, _HP - _HID))) for w in mlp_Ws[1:7]]
    w8 = jnp.pad(mlp_Ws[7], ((0, _HP - _HID), (0, 0)))
    bmid = [jnp.pad(b, (0, _HP - _HID)).reshape(1, _HP) for b in mlp_bs[:7]]
    b8 = mlp_bs[7].reshape(1, 1)

    rows = 512
    grid = _NP // rows
    full = lambda shape: pl.BlockSpec(shape, lambda i: (0, 0))

    operands = [z.reshape(_NP, 1), w1, bmid[0]]
    in_specs = [pl.BlockSpec((rows, 1), lambda i: (i, 0)), full((1, _HP)),
                full((1, _HP))]
    for wi, bi in zip(wmid, bmid[1:7]):
        operands += [wi, bi]
        in_specs += [full((_HP, _HP)), full((1, _HP))]
    operands += [w8, b8]
    in_specs += [full((_HP, 1)), full((1, 1))]

    out = pl.pallas_call(
        _mlp_body,
        grid=(grid,),
        in_specs=in_specs,
        out_specs=pl.BlockSpec((rows, 1), lambda i: (i, 0)),
        out_shape=jax.ShapeDtypeStruct((_NP, 1), f32),
    )(*operands)

    return out[:_N]


# speedup vs baseline: 72.6027x; 72.6027x over previous
import functools

import jax
import jax.numpy as jnp
from jax import lax
from jax.experimental import pallas as pl
from jax.experimental.pallas import tpu as pltpu
from jax.experimental.pallas import tpu_sc as plsc

_N = 10000
_E = 320000
_HID = 200
_HP = 256
_NP = 10240
_NG = _NP // 128
_NC = 2
_NS = 16
_NW = _NC * _NS
_EPW = _E // _NW
_L = 16

_MESH = plsc.VectorSubcoreMesh(
    core_axis_name="c", subcore_axis_name="s", num_cores=_NC, num_subcores=_NS
)


def _worker_id():
    return lax.axis_index("s") * _NC + lax.axis_index("c")


def _zero_vmem(ref, n):
    zeros = jnp.zeros((_L,), jnp.float32)

    def body(i, carry):
        ref[pl.ds(i * _L, _L)] = zeros
        return carry

    lax.fori_loop(0, n // _L, body, 0)


@functools.partial(
    pl.kernel,
    out_type=jax.ShapeDtypeStruct((_NW, _NP), jnp.float32),
    mesh=_MESH,
    scratch_types=[
        pltpu.VMEM((_EPW,), jnp.int32),
        pltpu.VMEM((_NP,), jnp.float32),
    ],
    compiler_params=pltpu.CompilerParams(needs_layout_passes=False),
)
def _sc_degree(dst_hbm, out_hbm, dst_v, acc_v):
    wid = _worker_id()
    pltpu.sync_copy(dst_hbm.at[pl.ds(wid * _EPW, _EPW)], dst_v)
    _zero_vmem(acc_v, _NP)
    ones = jnp.ones((_L,), jnp.float32)

    def body(i, carry):
        idx = dst_v[pl.ds(i * _L, _L)]
        plsc.addupdate_scatter(acc_v, [idx], ones)
        return carry

    lax.fori_loop(0, _EPW // _L, body, 0)
    pltpu.sync_copy(acc_v, out_hbm.at[wid])


@functools.partial(
    pl.kernel,
    out_type=jax.ShapeDtypeStruct((_NW, _NP), jnp.float32),
    mesh=_MESH,
    scratch_types=[
        pltpu.VMEM((_EPW,), jnp.int32),
        pltpu.VMEM((_EPW,), jnp.int32),
        pltpu.VMEM((_NP,), jnp.float32),
        pltpu.VMEM((_NP,), jnp.float32),
    ],
    compiler_params=pltpu.CompilerParams(needs_layout_passes=False),
)
def _sc_message(src_hbm, dst_hbm, u_hbm, out_hbm, src_v, dst_v, u_v, acc_v):
    wid = _worker_id()
    pltpu.sync_copy(u_hbm, u_v)
    pltpu.sync_copy(src_hbm.at[pl.ds(wid * _EPW, _EPW)], src_v)
    pltpu.sync_copy(dst_hbm.at[pl.ds(wid * _EPW, _EPW)], dst_v)
    _zero_vmem(acc_v, _NP)

    def body(i, carry):
        sidx = src_v[pl.ds(i * _L, _L)]
        didx = dst_v[pl.ds(i * _L, _L)]
        vals = plsc.load_gather(u_v, [sidx])
        plsc.addupdate_scatter(acc_v, [didx], vals)
        return carry

    lax.fori_loop(0, _EPW // _L, body, 0)
    pltpu.sync_copy(acc_v, out_hbm.at[wid])


def _matvec_body(x_ref, w_ref, h_ref):
    h_ref[...] = lax.dot(x_ref[...], w_ref[...], preferred_element_type=jnp.float32)


def _prep_body(degp_ref, h_ref, dinv_ref, u_ref):
    deg = jnp.sum(degp_ref[...], axis=0) + 1.0
    dinv = lax.rsqrt(deg)
    dinv_ref[...] = dinv
    u_ref[...] = dinv * h_ref[...]


def _combine_body(sp_ref, dinv_ref, u_ref, bg_ref, z_ref):
    s = jnp.sum(sp_ref[...], axis=0)
    z_ref[...] = dinv_ref[...] * (s + u_ref[...]) + bg_ref[0, 0]


def _mlp_body(z_ref, w1_ref, b1_ref, w2_ref, b2_ref, w3_ref, b3_ref, w4_ref,
              b4_ref, w5_ref, b5_ref, w6_ref, b6_ref, w7_ref, b7_ref, w8_ref,
              b8_ref, o_ref):
    a = jnp.maximum(z_ref[...] * w1_ref[...] + b1_ref[...], 0.0)
    for w_ref, b_ref in ((w2_ref, b2_ref), (w3_ref, b3_ref), (w4_ref, b4_ref),
                         (w5_ref, b5_ref), (w6_ref, b6_ref), (w7_ref, b7_ref)):
        a = lax.dot(a, w_ref[...], preferred_element_type=jnp.float32)
        a = jnp.maximum(a + b_ref[...], 0.0)
    logits = lax.dot(a, w8_ref[...], preferred_element_type=jnp.float32)
    o_ref[...] = jax.nn.sigmoid(logits + b8_ref[0, 0])


def kernel(x, edge_index, W_gcn, b_gcn, mlp_Ws, mlp_bs):
    f32 = jnp.float32
    src = edge_index[0]
    dst = edge_index[1]

    x_pad = jnp.pad(x, ((0, _NP - _N), (0, 0)))

    h = pl.pallas_call(
        _matvec_body,
        out_shape=jax.ShapeDtypeStruct((_NP, 1), f32),
    )(x_pad, W_gcn)

    degp = _sc_degree(dst)

    dinv, u = pl.pallas_call(
        _prep_body,
        out_shape=(
            jax.ShapeDtypeStruct((_NG, 128), f32),
            jax.ShapeDtypeStruct((_NG, 128), f32),
        ),
    )(degp.reshape(_NW, _NG, 128), h.reshape(_NG, 128))

    sp = _sc_message(src, dst, u.reshape(_NP))

    z = pl.pallas_call(
        _combine_body,
        out_shape=jax.ShapeDtypeStruct((_NG, 128), f32),
    )(sp.reshape(_NW, _NG, 128), dinv, u, b_gcn.reshape(1, 1).astype(f32))

    w1 = jnp.pad(mlp_Ws[0], ((0, 0), (0, _HP - _HID)))
    wmid = [jnp.pad(w, ((0, _HP - _HID), (0, _HP - _HID))) for w in mlp_Ws[1:7]]
    w8 = jnp.pad(mlp_Ws[7], ((0, _HP - _HID), (0, 0)))
    bmid = [jnp.pad(b, (0, _HP - _HID)).reshape(1, _HP) for b in mlp_bs[:7]]
    b8 = mlp_bs[7].reshape(1, 1)

    rows = 512
    grid = _NP // rows
    full = lambda shape: pl.BlockSpec(shape, lambda i: (0, 0))

    operands = [z.reshape(_NP, 1), w1, bmid[0]]
    in_specs = [pl.BlockSpec((rows, 1), lambda i: (i, 0)), full((1, _HP)),
                full((1, _HP))]
    for wi, bi in zip(wmid, bmid[1:7]):
        operands += [wi, bi]
        in_specs += [full((_HP, _HP)), full((1, _HP))]
    operands += [w8, b8]
    in_specs += [full((_HP, 1)), full((1, 1))]

    out = pl.pallas_call(
        _mlp_body,
        grid=(grid,),
        in_specs=in_specs,
        out_specs=pl.BlockSpec((rows, 1), lambda i: (i, 0)),
        out_shape=jax.ShapeDtypeStruct((_NP, 1), f32),
    )(*operands)

    return out[:_N]


# speedup vs baseline: 73.9456x; 1.0185x over previous
import functools

import jax
import jax.numpy as jnp
from jax import lax
from jax.experimental import pallas as pl
from jax.experimental.pallas import tpu as pltpu
from jax.experimental.pallas import tpu_sc as plsc

_N = 10000
_E = 320000
_HID = 200
_HP = 256
_NP = 10240
_NG = _NP // 128
_NC = 2
_NS = 16
_NW = _NC * _NS
_EPW = _E // _NW
_L = 16

_MESH = plsc.VectorSubcoreMesh(
    core_axis_name="c", subcore_axis_name="s", num_cores=_NC, num_subcores=_NS
)


def _worker_id():
    return lax.axis_index("s") * _NC + lax.axis_index("c")


def _zero_vmem(ref, n):
    zeros = jnp.zeros((_L,), jnp.float32)
    unroll = 8

    def body(i, carry):
        base = i * (unroll * _L)
        for k in range(unroll):
            ref[pl.ds(base + k * _L, _L)] = zeros
        return carry

    lax.fori_loop(0, n // (unroll * _L), body, 0)


@functools.partial(
    pl.kernel,
    out_type=jax.ShapeDtypeStruct((_NW, _NP), jnp.float32),
    mesh=_MESH,
    scratch_types=[
        pltpu.VMEM((_EPW,), jnp.int32),
        pltpu.VMEM((_NP,), jnp.float32),
    ],
    compiler_params=pltpu.CompilerParams(needs_layout_passes=False),
)
def _sc_degree(dst_hbm, out_hbm, dst_v, acc_v):
    wid = _worker_id()
    pltpu.sync_copy(dst_hbm.at[pl.ds(wid * _EPW, _EPW)], dst_v)
    _zero_vmem(acc_v, _NP)
    ones = jnp.ones((_L,), jnp.float32)

    def body(i, carry):
        base = i * (5 * _L)
        for k in range(5):
            idx = dst_v[pl.ds(base + k * _L, _L)]
            plsc.addupdate_scatter(acc_v, [idx], ones)
        return carry

    lax.fori_loop(0, _EPW // (5 * _L), body, 0)
    pltpu.sync_copy(acc_v, out_hbm.at[wid])


@functools.partial(
    pl.kernel,
    out_type=jax.ShapeDtypeStruct((_NW, _NP), jnp.float32),
    mesh=_MESH,
    scratch_types=[
        pltpu.VMEM((_EPW,), jnp.int32),
        pltpu.VMEM((_EPW,), jnp.int32),
        pltpu.VMEM((_NP,), jnp.float32),
        pltpu.VMEM((_NP,), jnp.float32),
    ],
    compiler_params=pltpu.CompilerParams(needs_layout_passes=False),
)
def _sc_message(src_hbm, dst_hbm, u_hbm, out_hbm, src_v, dst_v, u_v, acc_v):
    wid = _worker_id()
    pltpu.sync_copy(u_hbm, u_v)
    pltpu.sync_copy(src_hbm.at[pl.ds(wid * _EPW, _EPW)], src_v)
    pltpu.sync_copy(dst_hbm.at[pl.ds(wid * _EPW, _EPW)], dst_v)
    _zero_vmem(acc_v, _NP)

    def body(i, carry):
        base = i * (5 * _L)
        for k in range(5):
            sidx = src_v[pl.ds(base + k * _L, _L)]
            didx = dst_v[pl.ds(base + k * _L, _L)]
            vals = plsc.load_gather(u_v, [sidx])
            plsc.addupdate_scatter(acc_v, [didx], vals)
        return carry

    lax.fori_loop(0, _EPW // (5 * _L), body, 0)
    pltpu.sync_copy(acc_v, out_hbm.at[wid])


def _matvec_body(x_ref, w_ref, h_ref):
    h_ref[...] = lax.dot(x_ref[...], w_ref[...], preferred_element_type=jnp.float32)


def _prep_body(degp_ref, h_ref, dinv_ref, u_ref):
    deg = jnp.sum(degp_ref[...], axis=0) + 1.0
    dinv = lax.rsqrt(deg)
    dinv_ref[...] = dinv
    u_ref[...] = dinv * h_ref[...]


def _combine_body(sp_ref, dinv_ref, u_ref, bg_ref, z_ref):
    s = jnp.sum(sp_ref[...], axis=0)
    z_ref[...] = dinv_ref[...] * (s + u_ref[...]) + bg_ref[0, 0]


def _mlp_body(z_ref, w1_ref, b1_ref, w2_ref, b2_ref, w3_ref, b3_ref, w4_ref,
              b4_ref, w5_ref, b5_ref, w6_ref, b6_ref, w7_ref, b7_ref, w8_ref,
              b8_ref, o_ref):
    a = jnp.maximum(z_ref[...] * w1_ref[...] + b1_ref[...], 0.0)
    for w_ref, b_ref in ((w2_ref, b2_ref), (w3_ref, b3_ref), (w4_ref, b4_ref),
                         (w5_ref, b5_ref), (w6_ref, b6_ref), (w7_ref, b7_ref)):
        a = lax.dot(a, w_ref[...], preferred_element_type=jnp.float32)
        a = jnp.maximum(a + b_ref[...], 0.0)
    logits = lax.dot(a, w8_ref[...], preferred_element_type=jnp.float32)
    o_ref[...] = jax.nn.sigmoid(logits + b8_ref[0, 0])


def kernel(x, edge_index, W_gcn, b_gcn, mlp_Ws, mlp_bs):
    f32 = jnp.float32
    src = edge_index[0]
    dst = edge_index[1]

    x_pad = jnp.pad(x, ((0, _NP - _N), (0, 0)))

    h = pl.pallas_call(
        _matvec_body,
        out_shape=jax.ShapeDtypeStruct((_NP, 1), f32),
    )(x_pad, W_gcn)

    degp = _sc_degree(dst)

    dinv, u = pl.pallas_call(
        _prep_body,
        out_shape=(
            jax.ShapeDtypeStruct((_NG, 128), f32),
            jax.ShapeDtypeStruct((_NG, 128), f32),
        ),
    )(degp.reshape(_NW, _NG, 128), h.reshape(_NG, 128))

    sp = _sc_message(src, dst, u.reshape(_NP))

    z = pl.pallas_call(
        _combine_body,
        out_shape=jax.ShapeDtypeStruct((_NG, 128), f32),
    )(sp.reshape(_NW, _NG, 128), dinv, u, b_gcn.reshape(1, 1).astype(f32))

    w1 = jnp.pad(mlp_Ws[0], ((0, 0), (0, _HP - _HID)))
    wmid = [jnp.pad(w, ((0, _HP - _HID), (0, _HP - _HID))) for w in mlp_Ws[1:7]]
    w8 = jnp.pad(mlp_Ws[7], ((0, _HP - _HID), (0, 0)))
    bmid = [jnp.pad(b, (0, _HP - _HID)).reshape(1, _HP) for b in mlp_bs[:7]]
    b8 = mlp_bs[7].reshape(1, 1)

    rows = 512
    grid = _NP // rows
    full = lambda shape: pl.BlockSpec(shape, lambda i: (0, 0))

    operands = [z.reshape(_NP, 1), w1, bmid[0]]
    in_specs = [pl.BlockSpec((rows, 1), lambda i: (i, 0)), full((1, _HP)),
                full((1, _HP))]
    for wi, bi in zip(wmid, bmid[1:7]):
        operands += [wi, bi]
        in_specs += [full((_HP, _HP)), full((1, _HP))]
    operands += [w8, b8]
    in_specs += [full((_HP, 1)), full((1, 1))]

    out = pl.pallas_call(
        _mlp_body,
        grid=(grid,),
        in_specs=in_specs,
        out_specs=pl.BlockSpec((rows, 1), lambda i: (i, 0)),
        out_shape=jax.ShapeDtypeStruct((_NP, 1), f32),
    )(*operands)

    return out[:_N]
